# baseline (speedup 1.0000x reference)
import functools

import jax
import jax.numpy as jnp
from jax import lax
from jax.experimental import pallas as pl
from jax.experimental.pallas import tpu as pltpu
from jax.experimental.pallas import tpu_sc as plsc

NC = 2
NS = 16
L = 16
NW = NC * NS
_NSTAT = 5

_SQRT2 = 1.4142135623730951
_LN2 = 0.6931471805599453


def _vlog(x):
    bits = lax.bitcast_convert_type(x, jnp.int32)
    e = lax.shift_right_logical(bits, 23) - 127
    m_bits = (bits & jnp.int32(0x7FFFFF)) | jnp.int32(0x3F800000)
    m = lax.bitcast_convert_type(m_bits, jnp.float32)
    big = m > _SQRT2
    m = jnp.where(big, 0.5 * m, m)
    e = e + jnp.where(big, 1, 0)
    ef = e.astype(jnp.float32)
    t = (m - 1.0) / (m + 1.0)
    t2 = t * t
    p = jnp.float32(1.0 / 7.0)
    p = p * t2 + jnp.float32(1.0 / 5.0)
    p = p * t2 + jnp.float32(1.0 / 3.0)
    p = p * t2 + 1.0
    return ef * jnp.float32(_LN2) + 2.0 * t * p


def _make_sc_kernel(B, D, interpret=False):
    bpw = B // NW

    def body(cw_hbm, xw_hbm, co_hbm, w_hbm, cemb_hbm, xemb_hbm,
             cbias_hbm, xbias_hbm, out_hbm,
             idx_cv, idx_xv, rows_c, rows_x, co_v, w_v,
             bc_v, bx_v, part_v, sem, semb):
        wid = lax.axis_index("c") * NS + lax.axis_index("s")
        base = wid * bpw
        pltpu.sync_copy(cw_hbm.at[pl.ds(base, bpw)], idx_cv)
        pltpu.sync_copy(xw_hbm.at[pl.ds(base, bpw)], idx_xv)
        pltpu.sync_copy(co_hbm.at[pl.ds(base, bpw)], co_v)
        pltpu.sync_copy(w_hbm.at[pl.ds(base, bpw)], w_v)

        def fetch_group(g, _):
            gb = g * L
            vecc = idx_cv[pl.ds(gb, L)]
            vecx = idx_xv[pl.ds(gb, L)]
            for k in range(L):
                vc = vecc[k]
                vx = vecx[k]
                pltpu.async_copy(cemb_hbm.at[pl.ds(vc, 1)],
                                 rows_c.at[pl.ds(gb + k, 1)], sem)
                pltpu.async_copy(xemb_hbm.at[pl.ds(vx, 1)],
                                 rows_x.at[pl.ds(gb + k, 1)], sem)
                pltpu.async_copy(cbias_hbm.at[pl.ds(vc, 1)],
                                 bc_v.at[pl.ds(gb + k, 1)], semb)
                pltpu.async_copy(xbias_hbm.at[pl.ds(vx, 1)],
                                 bx_v.at[pl.ds(gb + k, 1)], semb)
            pltpu.make_async_copy(cemb_hbm.at[pl.ds(0, L)],
                                  rows_c.at[pl.ds(gb, L)], sem).wait()
            pltpu.make_async_copy(xemb_hbm.at[pl.ds(0, L)],
                                  rows_x.at[pl.ds(gb, L)], sem).wait()
            pltpu.make_async_copy(cbias_hbm.at[pl.ds(0, L)],
                                  bc_v.at[pl.ds(gb, L)], semb).wait()
            pltpu.make_async_copy(xbias_hbm.at[pl.ds(0, L)],
                                  bx_v.at[pl.ds(gb, L)], semb).wait()
            return 0

        lax.fori_loop(0, bpw // L, fetch_group, 0)

        zero = jnp.zeros((L,), jnp.float32)
        lane = lax.iota(jnp.int32, L)
        s_wa2 = zero
        s_wa = zero
        s_w = zero
        s_b = zero
        s_b2 = zero
        for g in range(bpw // L):
            row = g * L + lane

            def dot_step(dd, acc, row=row):
                col = jnp.full((L,), dd, jnp.int32)
                av = plsc.load_gather(rows_c, [row, col])
                bv = plsc.load_gather(rows_x, [row, col])
                return acc + av * bv

            sim = lax.fori_loop(0, D, dot_step, zero)
            cog = co_v[pl.ds(g * L, L)]
            wg = w_v[pl.ds(g * L, L)]
            a = sim - _vlog(cog)
            s_wa2 = s_wa2 + wg * a * a
            s_wa = s_wa + wg * a
            s_w = s_w + wg
            zcol = jnp.zeros((L,), jnp.int32)
            bg = (plsc.load_gather(bc_v, [row, zcol])
                  + plsc.load_gather(bx_v, [row, zcol]))
            s_b = s_b + bg
            s_b2 = s_b2 + bg * bg

        part_v[pl.ds(0 * L, L)] = s_wa2
        part_v[pl.ds(1 * L, L)] = s_wa
        part_v[pl.ds(2 * L, L)] = s_w
        part_v[pl.ds(3 * L, L)] = s_b
        part_v[pl.ds(4 * L, L)] = s_b2
        pltpu.sync_copy(part_v, out_hbm.at[pl.ds(wid * _NSTAT * L, _NSTAT * L)])

    return pl.kernel(
        body,
        out_type=jax.ShapeDtypeStruct((NW * _NSTAT * L,), jnp.float32),
        mesh=plsc.VectorSubcoreMesh(core_axis_name="c", subcore_axis_name="s",
                                    num_cores=NC),
        scratch_types=[
            pltpu.VMEM((bpw,), jnp.int32),
            pltpu.VMEM((bpw,), jnp.int32),
            pltpu.VMEM((bpw, D), jnp.float32),
            pltpu.VMEM((bpw, D), jnp.float32),
            pltpu.VMEM((bpw,), jnp.float32),
            pltpu.VMEM((bpw,), jnp.float32),
            pltpu.VMEM((bpw, 1), jnp.float32),
            pltpu.VMEM((bpw, 1), jnp.float32),
            pltpu.VMEM((_NSTAT * L,), jnp.float32),
            pltpu.SemaphoreType.DMA,
            pltpu.SemaphoreType.DMA,
        ],
        compiler_params=pltpu.CompilerParams(needs_layout_passes=False),
        interpret=interpret,
    )


def kernel(center_word, context_word, co_mat_val, weight_mat_val,
           center_embedding, context_embedding, center_bias, context_bias):
    B = center_word.shape[0]
    V, D = center_embedding.shape
    cw = center_word.astype(jnp.int32)
    xw = context_word.astype(jnp.int32)
    co = co_mat_val.astype(jnp.float32)
    wv = weight_mat_val.astype(jnp.float32)
    cb = center_bias.astype(jnp.float32)
    xb = context_bias.astype(jnp.float32)

    partials = _make_sc_kernel(B, D)(
        cw, xw, co, wv, center_embedding, context_embedding, cb, xb)
    p = partials.reshape(NW, _NSTAT, L).sum(axis=(0, 2))
    s_wa2, s_wa, s_w, s_b, s_b2 = p[0], p[1], p[2], p[3], p[4]
    return 0.5 * (B * s_wa2 + 2.0 * s_b * s_wa + s_b2 * s_w)


# speedup vs baseline: 1.4358x; 1.4358x over previous
import functools

import jax
import jax.numpy as jnp
from jax import lax
from jax.experimental import pallas as pl
from jax.experimental.pallas import tpu as pltpu
from jax.experimental.pallas import tpu_sc as plsc

NC = 2
NS = 16
L = 16
NW = NC * NS
_NSTAT = 5

_SQRT2 = 1.4142135623730951
_LN2 = 0.6931471805599453


def _vlog(x):
    bits = lax.bitcast_convert_type(x, jnp.int32)
    e = lax.shift_right_logical(bits, 23) - 127
    m_bits = (bits & jnp.int32(0x7FFFFF)) | jnp.int32(0x3F800000)
    m = lax.bitcast_convert_type(m_bits, jnp.float32)
    big = m > _SQRT2
    m = jnp.where(big, 0.5 * m, m)
    e = e + jnp.where(big, 1, 0)
    ef = e.astype(jnp.float32)
    t = (m - 1.0) / (m + 1.0)
    t2 = t * t
    p = jnp.float32(1.0 / 7.0)
    p = p * t2 + jnp.float32(1.0 / 5.0)
    p = p * t2 + jnp.float32(1.0 / 3.0)
    p = p * t2 + 1.0
    return ef * jnp.float32(_LN2) + 2.0 * t * p


def _make_sc_kernel(B, D, interpret=False):
    bpw = B // NW

    def body(cw_hbm, xw_hbm, co_hbm, w_hbm, cemb_hbm, xemb_hbm,
             cbias_hbm, xbias_hbm, out_hbm,
             idx_cv, idx_xv, rows_c, rows_x, co_v, w_v,
             bc_v, bx_v, part_v, sem, semb):
        wid = lax.axis_index("c") * NS + lax.axis_index("s")
        base = wid * bpw
        pltpu.sync_copy(cw_hbm.at[pl.ds(base, bpw)], idx_cv)
        pltpu.sync_copy(xw_hbm.at[pl.ds(base, bpw)], idx_xv)
        pltpu.sync_copy(co_hbm.at[pl.ds(base, bpw)], co_v)
        pltpu.sync_copy(w_hbm.at[pl.ds(base, bpw)], w_v)
        cpb1 = pltpu.async_copy(cbias_hbm.at[idx_cv], bc_v, semb)
        cpb2 = pltpu.async_copy(xbias_hbm.at[idx_xv], bx_v, semb)

        def fetch_group(g, _):
            gb = g * L
            vecc = idx_cv[pl.ds(gb, L)]
            vecx = idx_xv[pl.ds(gb, L)]
            for k in range(L):
                vc = vecc[k]
                vx = vecx[k]
                pltpu.async_copy(cemb_hbm.at[pl.ds(vc, 1)],
                                 rows_c.at[pl.ds(gb + k, 1)], sem)
                pltpu.async_copy(xemb_hbm.at[pl.ds(vx, 1)],
                                 rows_x.at[pl.ds(gb + k, 1)], sem)
            return 0

        lax.fori_loop(0, bpw // L, fetch_group, 0)
        pltpu.make_async_copy(cemb_hbm.at[pl.ds(0, bpw)], rows_c, sem).wait()
        pltpu.make_async_copy(xemb_hbm.at[pl.ds(0, bpw)], rows_x, sem).wait()
        cpb1.wait()
        cpb2.wait()

        zero = jnp.zeros((L,), jnp.float32)
        lane = lax.iota(jnp.int32, L)
        s_wa2 = zero
        s_wa = zero
        s_w = zero
        s_b = zero
        s_b2 = zero
        for g in range(bpw // L):
            row = g * L + lane

            def dot_step(dd, acc, row=row):
                col = jnp.full((L,), dd, jnp.int32)
                av = plsc.load_gather(rows_c, [row, col])
                bv = plsc.load_gather(rows_x, [row, col])
                return acc + av * bv

            sim = lax.fori_loop(0, D, dot_step, zero)
            cog = co_v[pl.ds(g * L, L)]
            wg = w_v[pl.ds(g * L, L)]
            a = sim - _vlog(cog)
            s_wa2 = s_wa2 + wg * a * a
            s_wa = s_wa + wg * a
            s_w = s_w + wg
            bg = bc_v[pl.ds(g * L, L)] + bx_v[pl.ds(g * L, L)]
            s_b = s_b + bg
            s_b2 = s_b2 + bg * bg

        part_v[pl.ds(0 * L, L)] = s_wa2
        part_v[pl.ds(1 * L, L)] = s_wa
        part_v[pl.ds(2 * L, L)] = s_w
        part_v[pl.ds(3 * L, L)] = s_b
        part_v[pl.ds(4 * L, L)] = s_b2
        pltpu.sync_copy(part_v, out_hbm.at[pl.ds(wid * _NSTAT * L, _NSTAT * L)])

    return pl.kernel(
        body,
        out_type=jax.ShapeDtypeStruct((NW * _NSTAT * L,), jnp.float32),
        mesh=plsc.VectorSubcoreMesh(core_axis_name="c", subcore_axis_name="s",
                                    num_cores=NC),
        scratch_types=[
            pltpu.VMEM((bpw,), jnp.int32),
            pltpu.VMEM((bpw,), jnp.int32),
            pltpu.VMEM((bpw, D), jnp.float32),
            pltpu.VMEM((bpw, D), jnp.float32),
            pltpu.VMEM((bpw,), jnp.float32),
            pltpu.VMEM((bpw,), jnp.float32),
            pltpu.VMEM((bpw,), jnp.float32),
            pltpu.VMEM((bpw,), jnp.float32),
            pltpu.VMEM((_NSTAT * L,), jnp.float32),
            pltpu.SemaphoreType.DMA,
            pltpu.SemaphoreType.DMA,
        ],
        compiler_params=pltpu.CompilerParams(needs_layout_passes=False),
        interpret=interpret,
    )


def kernel(center_word, context_word, co_mat_val, weight_mat_val,
           center_embedding, context_embedding, center_bias, context_bias):
    B = center_word.shape[0]
    V, D = center_embedding.shape
    cw = center_word.astype(jnp.int32)
    xw = context_word.astype(jnp.int32)
    co = co_mat_val.astype(jnp.float32)
    wv = weight_mat_val.astype(jnp.float32)
    cb = center_bias.reshape((V,)).astype(jnp.float32)
    xb = context_bias.reshape((V,)).astype(jnp.float32)

    partials = _make_sc_kernel(B, D)(
        cw, xw, co, wv, center_embedding, context_embedding, cb, xb)
    p = partials.reshape(NW, _NSTAT, L).sum(axis=(0, 2))
    s_wa2, s_wa, s_w, s_b, s_b2 = p[0], p[1], p[2], p[3], p[4]
    return 0.5 * (B * s_wa2 + 2.0 * s_b * s_wa + s_b2 * s_w)
